# Initial kernel scaffold; baseline (speedup 1.0000x reference)
import jax
import jax.numpy as jnp
from jax.experimental import pallas as pl


def kernel(vid, Wq, bq, Wk, bk, Wv, bv, Wp, bp):
    raise NotImplementedError("write your pallas kernel here")


# speedup vs baseline: 28.6989x; 28.6989x over previous
import numpy as np
import jax
import jax.numpy as jnp
from jax.experimental import pallas as pl
from jax.experimental.pallas import tpu as pltpu

_B, _T, _HEADS, _DH = 1, 2, 3, 32
_DIM = _HEADS * _DH
_H = 96
_W = 96
_PS = 7
_STRIDE0 = 4
_K = 10
_SCALE = 1.0 / float(np.sqrt(_DH))
_nH = (_H - 1) // _STRIDE0 + 1
_nW = (_W - 1) // _STRIDE0 + 1
_Qn = _T * _nH * _nW
_Kn = _T * _H * _W
_QB = 128
_NQB = _Qn // _QB


def _np_reflect(idx, n):
    idx = np.where(idx < 0, -idx, idx)
    idx = np.where(idx >= n, 2 * (n - 1) - idx, idx)
    return idx


def _zcount():
    ci = np.minimum(np.arange(_nH) * _STRIDE0, _H - 1)
    cj = np.minimum(np.arange(_nW) * _STRIDE0, _W - 1)
    off = np.arange(_PS) - _PS // 2
    fi = _np_reflect(ci[:, None] + off[None, :], _H)
    fj = _np_reflect(cj[:, None] + off[None, :], _W)
    flat = (np.arange(_T)[:, None, None, None, None] * (_H * _W)
            + fi[None, :, None, :, None] * _W
            + fj[None, None, :, None, :])
    z = np.zeros((_T * _H * _W,), np.float32)
    np.add.at(z, flat.ravel(), 1.0)
    return z.reshape(_T, _H, _W, 1)


_ZCOUNT = _zcount()


def _qkv_kernel(x_ref, w_ref, b_ref, o_ref):
    o_ref[...] = jnp.dot(x_ref[...], w_ref[...],
                         preferred_element_type=jnp.float32) + b_ref[...]


def _topk_kernel(qg_ref, kt_ref, w_ref, i_ref):
    d = jnp.dot(qg_ref[0], kt_ref[0], preferred_element_type=jnp.float32)
    iota = jax.lax.broadcasted_iota(jnp.int32, d.shape, 1)
    vals = []
    idxs = []
    for _ in range(_K):
        m = jnp.max(d, axis=1, keepdims=True)
        am = jnp.min(jnp.where(d == m, iota, jnp.int32(2 ** 30)),
                     axis=1, keepdims=True)
        vals.append(m)
        idxs.append(am)
        d = jnp.where(iota == am, jnp.float32(-1e30), d)
    v10 = jnp.concatenate(vals, axis=1) * jnp.float32(_SCALE)
    m10 = jnp.max(v10, axis=1, keepdims=True)
    e10 = jnp.exp(v10 - m10)
    w_ref[0] = e10 / jnp.sum(e10, axis=1, keepdims=True)
    i_ref[0] = jnp.concatenate(idxs, axis=1)


def _gather_kernel(t_sref, i_sref, j_sref, w_sref, vpad_ref, o_ref):
    h = pl.program_id(0)
    b = pl.program_id(1)
    base = (h * _Qn + b * _QB) * _K

    def qbody(q, _):
        def kbody(kk, acc):
            idx = base + q * _K + kk
            tt = t_sref[idx]
            ii = i_sref[idx]
            jj = j_sref[idx]
            patch = vpad_ref[0, tt, pl.ds(ii, _PS), pl.ds(jj, _PS), :]
            return acc + w_sref[idx] * patch

        acc = jax.lax.fori_loop(
            0, _K, kbody, jnp.zeros((_PS, _PS, _DH), jnp.float32))
        o_ref[0, pl.ds(q, 1)] = acc[None]
        return 0

    jax.lax.fori_loop(0, _QB, qbody, 0)


def _proj_kernel(x_ref, w_ref, b_ref, o_ref):
    o_ref[...] = jnp.dot(x_ref[...], w_ref[...],
                         preferred_element_type=jnp.float32) + b_ref[...]


def _fold_kernel(g_ref, z_ref, o_ref, acc_ref):
    ga = pl.program_id(0)
    gb = pl.program_id(1)

    @pl.when(jnp.logical_and(ga == 0, gb == 0))
    def _():
        acc_ref[...] = jnp.zeros_like(acc_ref)

    blk = g_ref[0, 0]
    zv = jnp.zeros((_T, 12, 12, _PS, 1, _DIM), jnp.float32)
    blk = jnp.concatenate([blk, zv], axis=4)
    zu = jnp.zeros((_T, 12, 12, 1, _PS + 1, _DIM), jnp.float32)
    blk = jnp.concatenate([blk, zu], axis=3)
    arr = blk.transpose(0, 1, 3, 2, 4, 5).reshape(_T, 96, 96, _DIM)
    r0 = 4 * ga
    c0 = 4 * gb
    cur = acc_ref[:, pl.ds(r0, 96), pl.ds(c0, 96), :]
    acc_ref[:, pl.ds(r0, 96), pl.ds(c0, 96), :] = cur + arr

    @pl.when(jnp.logical_and(ga == 1, gb == 1))
    def _():
        a = acc_ref[...]
        core = a[:, 3:99, :, :]
        top = jnp.concatenate(
            [a[:, 2:3], a[:, 1:2], a[:, 0:1]], axis=1)
        bot = jnp.concatenate(
            [a[:, 101:102], a[:, 100:101], a[:, 99:100]], axis=1)
        zt = jnp.zeros((_T, 1, 102, _DIM), jnp.float32)
        zb = jnp.zeros((_T, 92, 102, _DIM), jnp.float32)
        core = core + jnp.concatenate([zt, top, zb], axis=1)
        zt2 = jnp.zeros((_T, 92, 102, _DIM), jnp.float32)
        zb2 = jnp.zeros((_T, 1, 102, _DIM), jnp.float32)
        core = core + jnp.concatenate([zt2, bot, zb2], axis=1)
        ccore = core[:, :, 3:99, :]
        left = jnp.concatenate(
            [core[:, :, 2:3], core[:, :, 1:2], core[:, :, 0:1]], axis=2)
        right = jnp.concatenate(
            [core[:, :, 101:102], core[:, :, 100:101], core[:, :, 99:100]],
            axis=2)
        zl = jnp.zeros((_T, 96, 1, _DIM), jnp.float32)
        zr = jnp.zeros((_T, 96, 92, _DIM), jnp.float32)
        ccore = ccore + jnp.concatenate([zl, left, zr], axis=2)
        zl2 = jnp.zeros((_T, 96, 92, _DIM), jnp.float32)
        zr2 = jnp.zeros((_T, 96, 1, _DIM), jnp.float32)
        ccore = ccore + jnp.concatenate([zl2, right, zr2], axis=2)
        o_ref[...] = ccore / z_ref[...]


def kernel(vid, Wq, bq, Wk, bk, Wv, bv, Wp, bp):
    x = vid.reshape(_T, _DIM, _H, _W).transpose(0, 2, 3, 1).reshape(_Kn, _DIM)

    wcat = jnp.concatenate([Wq.T, Wk.T, Wv.T], axis=1)
    bcat = jnp.concatenate([bq, bk, bv]).reshape(1, 3 * _DIM)
    rb = _Kn // 8
    qkv = pl.pallas_call(
        _qkv_kernel,
        grid=(8,),
        in_specs=[
            pl.BlockSpec((rb, _DIM), lambda r: (r, 0)),
            pl.BlockSpec((_DIM, 3 * _DIM), lambda r: (0, 0)),
            pl.BlockSpec((1, 3 * _DIM), lambda r: (0, 0)),
        ],
        out_specs=pl.BlockSpec((rb, 3 * _DIM), lambda r: (r, 0)),
        out_shape=jax.ShapeDtypeStruct((_Kn, 3 * _DIM), jnp.float32),
    )(x, wcat, bcat)

    q = qkv[:, :_DIM]
    k = qkv[:, _DIM:2 * _DIM]
    v = qkv[:, 2 * _DIM:]

    qg = (q.reshape(_T, _H, _W, _HEADS, _DH)[:, ::_STRIDE0, ::_STRIDE0]
          .transpose(3, 0, 1, 2, 4).reshape(_HEADS, _Qn, _DH))
    kt = (k.reshape(_T, _H, _W, _HEADS, _DH)
          .transpose(3, 4, 0, 1, 2).reshape(_HEADS, _DH, _Kn))
    vh = (v.reshape(_T, _H, _W, _HEADS, _DH)
          .transpose(3, 0, 1, 2, 4))
    vpad = jnp.pad(vh, ((0, 0), (0, 0), (3, 3), (3, 3), (0, 0)),
                   mode='reflect')

    w10, i10 = pl.pallas_call(
        _topk_kernel,
        grid=(_HEADS, _NQB),
        in_specs=[
            pl.BlockSpec((1, _QB, _DH), lambda h, b: (h, b, 0)),
            pl.BlockSpec((1, _DH, _Kn), lambda h, b: (h, 0, 0)),
        ],
        out_specs=[
            pl.BlockSpec((1, _QB, _K), lambda h, b: (h, b, 0)),
            pl.BlockSpec((1, _QB, _K), lambda h, b: (h, b, 0)),
        ],
        out_shape=[
            jax.ShapeDtypeStruct((_HEADS, _Qn, _K), jnp.float32),
            jax.ShapeDtypeStruct((_HEADS, _Qn, _K), jnp.int32),
        ],
    )(qg, kt)

    ti = i10 // (_H * _W)
    rem = i10 - ti * (_H * _W)
    ii = rem // _W
    jj = rem - ii * _W
    tf = ti.reshape(-1)
    ifl = ii.reshape(-1)
    jf = jj.reshape(-1)
    wf = w10.reshape(-1)

    agg = pl.pallas_call(
        _gather_kernel,
        grid_spec=pltpu.PrefetchScalarGridSpec(
            num_scalar_prefetch=4,
            grid=(_HEADS, _NQB),
            in_specs=[
                pl.BlockSpec((1, _T, _H + 6, _W + 6, _DH),
                             lambda h, b, *_: (h, 0, 0, 0, 0)),
            ],
            out_specs=pl.BlockSpec((1, _QB, _PS, _PS, _DH),
                                   lambda h, b, *_: (h, b, 0, 0, 0)),
        ),
        out_shape=jax.ShapeDtypeStruct((_HEADS, _Qn, _PS, _PS, _DH),
                                       jnp.float32),
    )(tf, ifl, jf, wf, vpad)

    patches = (agg.transpose(1, 2, 3, 0, 4)
               .reshape(_Qn * _PS * _PS, _DIM))

    pr = _Qn * _PS * _PS // 8
    proj = pl.pallas_call(
        _proj_kernel,
        grid=(8,),
        in_specs=[
            pl.BlockSpec((pr, _DIM), lambda r: (r, 0)),
            pl.BlockSpec((_DIM, _DIM), lambda r: (0, 0)),
            pl.BlockSpec((1, _DIM), lambda r: (0, 0)),
        ],
        out_specs=pl.BlockSpec((pr, _DIM), lambda r: (r, 0)),
        out_shape=jax.ShapeDtypeStruct((_Qn * _PS * _PS, _DIM), jnp.float32),
    )(patches, Wp.T, bp.reshape(1, _DIM))

    g = (proj.reshape(_T, 12, 2, 12, 2, _PS, _PS, _DIM)
         .transpose(2, 4, 0, 1, 3, 5, 6, 7))

    z = jnp.asarray(_ZCOUNT)
    out = pl.pallas_call(
        _fold_kernel,
        grid=(2, 2),
        in_specs=[
            pl.BlockSpec((1, 1, _T, 12, 12, _PS, _PS, _DIM),
                         lambda ga, gb: (ga, gb, 0, 0, 0, 0, 0, 0)),
            pl.BlockSpec((_T, _H, _W, 1), lambda ga, gb: (0, 0, 0, 0)),
        ],
        out_specs=pl.BlockSpec((_T, _H, _W, _DIM),
                               lambda ga, gb: (0, 0, 0, 0)),
        out_shape=jax.ShapeDtypeStruct((_T, _H, _W, _DIM), jnp.float32),
        scratch_shapes=[pltpu.VMEM((_T, _H + 6, _W + 6, _DIM), jnp.float32)],
    )(g, z)

    return out.transpose(0, 3, 1, 2).reshape(_B, _T, _DIM, _H, _W)
